# baseline (speedup 1.0000x reference)
import jax
import jax.numpy as jnp
from jax import lax
from jax.experimental import pallas as pl
from jax.experimental.pallas import tpu as pltpu
from jax.experimental.pallas import tpu_sc as plsc

N_IN = 600000
N_OUT = 120000
F = 32
K = 9
R = K * F
NB = 30
BLK_SEG = N_OUT // NB
TILES = 16
NCORES = 2
BLK_PER_CORE = NB // NCORES
ROWS_PT = R // TILES
CHUNK = 128
GROUPS = CHUNK // 16
ACC_W = R * BLK_SEG
DUMPC = ACC_W
SH_W = ACC_W + (F - 1) * BLK_SEG + 16


def _sc_body(feat_hbm, dt_hbm, times_hbm, nrate_hbm, kid_hbm, seg_hbm,
             bounds_hbm, zeros_hbm, out_hbm,
             acc, times_v, feat_v, dt_v, seg_v, kid_v, vals_v, idx_v,
             bounds_v, nrate_v, zbuf, sem):
    c = lax.axis_index("c")
    t = lax.axis_index("s")
    pltpu.sync_copy(bounds_hbm, bounds_v)
    pltpu.sync_copy(nrate_hbm, nrate_v)
    pltpu.sync_copy(zeros_hbm, zbuf)
    nrate_lo = nrate_v[pl.ds(0, 16)]
    nrate_hi = nrate_v[pl.ds(16, 16)]
    iota = lax.broadcasted_iota(jnp.int32, (16,), 0)

    def block_body(j, carry):
        b = c * BLK_PER_CORE + j
        base = b * BLK_SEG
        for u in range(ROWS_PT):
            r = t * ROWS_PT + u
            pltpu.sync_copy(zbuf, acc.at[pl.ds(r * BLK_SEG, BLK_SEG)])
        pltpu.sync_copy(times_hbm.at[pl.ds(base, BLK_SEG)], times_v)
        plsc.subcore_barrier()
        bv = bounds_v[pl.ds(b, 16)]
        lo = bv[0]
        hi = bv[1]
        n = hi - lo
        sh = (n + TILES - 1) // TILES
        a = lo + t * sh
        bb = jnp.minimum(a + sh, hi)
        start0 = (a // 8) * 8
        nc = jnp.maximum((bb - start0 + CHUNK - 1) // CHUNK, 0)

        def chunk_body(ci, carry2):
            cs = jnp.minimum(start0 + ci * CHUNK, N_IN - CHUNK)
            lo_c = jnp.maximum(a, start0 + ci * CHUNK)
            hi_c = jnp.minimum(bb, start0 + ci * CHUNK + CHUNK)
            cp1 = pltpu.async_copy(feat_hbm.at[pl.ds(cs, CHUNK)], feat_v, sem)
            cp2 = pltpu.async_copy(dt_hbm.at[pl.ds(cs, CHUNK)], dt_v, sem)
            cp3 = pltpu.async_copy(seg_hbm.at[pl.ds(cs, CHUNK)], seg_v, sem)
            cp4 = pltpu.async_copy(kid_hbm.at[pl.ds(cs, CHUNK)], kid_v, sem)
            cp1.wait(); cp2.wait(); cp3.wait(); cp4.wait()
            deltas = []
            bases = []
            for g in range(GROUPS):
                off = g * 16
                sg = seg_v[pl.ds(off, 16)]
                kd = kid_v[pl.ds(off, 16)]
                dtv = dt_v[pl.ds(off, 16)]
                relc = jnp.clip(sg - base, 0, BLK_SEG - 1)
                tv = plsc.load_gather(times_v, [relc])
                deltas.append(tv - dtv)
                gi = cs + off + iota
                valid = (gi >= lo_c) & (gi < hi_c)
                bases.append(
                    jnp.where(valid, kd * (F * BLK_SEG) + relc, DUMPC))
            for f in range(F):
                rf = nrate_lo[f] if f < 16 else nrate_hi[f - 16]
                colf = jnp.full((16,), f, jnp.int32)
                for g in range(GROUPS):
                    off = g * 16
                    fv = plsc.load_gather(feat_v, [off + iota, colf])
                    val = fv * jnp.exp(deltas[g] * rf)
                    vals_v[f, pl.ds(off, 16)] = val
                    idx_v[f, pl.ds(off, 16)] = bases[g] + (f * BLK_SEG)
            cps = []
            for f in range(F):
                cps.append(pltpu.async_copy(
                    vals_v.at[f], acc.at[idx_v.at[f]], sem, add=True))
            for cp in cps:
                cp.wait()
            return carry2

        lax.fori_loop(0, nc, chunk_body, 0)
        plsc.subcore_barrier()
        for u in range(ROWS_PT):
            r = t * ROWS_PT + u
            pltpu.sync_copy(acc.at[pl.ds(r * BLK_SEG, BLK_SEG)],
                            out_hbm.at[pl.ds(r * N_OUT + base, BLK_SEG)])
        return carry

    lax.fori_loop(0, BLK_PER_CORE, block_body, 0)


def kernel(features, dt, times_out, decay_rate, successor_kernel_ids,
           segment_ids_out):
    nrate = -jax.nn.softplus(decay_rate).astype(jnp.float32)
    starts = (jnp.arange(NB + 1, dtype=jnp.int32) * BLK_SEG)
    bounds = jnp.searchsorted(segment_ids_out, starts,
                              method="compare_all").astype(jnp.int32)
    bounds48 = jnp.concatenate(
        [bounds, jnp.full((48 - (NB + 1),), N_IN, dtype=jnp.int32)])
    zeros_c = jnp.zeros((BLK_SEG,), dtype=jnp.float32)

    kern = pl.kernel(
        _sc_body,
        out_type=jax.ShapeDtypeStruct((R * N_OUT,), jnp.float32),
        mesh=plsc.VectorSubcoreMesh(core_axis_name="c", subcore_axis_name="s"),
        scratch_types=[
            pltpu.VMEM_SHARED((SH_W,), jnp.float32),
            pltpu.VMEM((BLK_SEG,), jnp.float32),
            pltpu.VMEM((CHUNK, F), jnp.float32),
            pltpu.VMEM((CHUNK,), jnp.float32),
            pltpu.VMEM((CHUNK,), jnp.int32),
            pltpu.VMEM((CHUNK,), jnp.int32),
            pltpu.VMEM((F, CHUNK), jnp.float32),
            pltpu.VMEM((F, CHUNK), jnp.int32),
            pltpu.VMEM((48,), jnp.int32),
            pltpu.VMEM((F,), jnp.float32),
            pltpu.VMEM((BLK_SEG,), jnp.float32),
            pltpu.SemaphoreType.DMA,
        ],
        compiler_params=pltpu.CompilerParams(
            needs_layout_passes=False, use_tc_tiling_on_sc=False),
    )
    out1d = kern(features, dt, times_out, nrate, successor_kernel_ids,
                 segment_ids_out, bounds48, zeros_c)
    return out1d.reshape(K, F, N_OUT).transpose(2, 0, 1)
